# Initial kernel scaffold; baseline (speedup 1.0000x reference)
import jax
import jax.numpy as jnp
from jax.experimental import pallas as pl


def kernel(features, msk, samp_bias1, samp_bias2, W1, W2, Wd, bd, neigh, nodes, perm):
    raise NotImplementedError("write your pallas kernel here")


# speedup vs baseline: 3.5334x; 3.5334x over previous
import functools

import jax
import jax.numpy as jnp
from jax import lax
from jax.experimental import pallas as pl
from jax.experimental.pallas import tpu as pltpu
from jax.experimental.pallas import tpu_sc as plsc


def _sc_geometry():
    try:
        info = plsc.get_sparse_core_info()
        return info.num_cores, info.num_subcores
    except Exception:
        return 2, 16


def _row_gather(table, idx, n_chunks):
    NC, NS = _sc_geometry()
    NW = NC * NS
    M = idx.shape[0]
    T, D = table.shape
    per_w = M // NW
    nb = per_w // n_chunks
    mesh = plsc.VectorSubcoreMesh(core_axis_name="c", subcore_axis_name="s",
                                  num_cores=NC, num_subcores=NS)

    @functools.partial(
        pl.kernel, mesh=mesh,
        out_type=jax.ShapeDtypeStruct((M, D), table.dtype),
        scratch_types=[
            pltpu.VMEM((nb,), jnp.int32),
            pltpu.VMEM((nb, D), table.dtype),
            pltpu.SemaphoreType.DMA,
        ],
    )
    def k(table_hbm, idx_hbm, out_hbm, idx_v, rows_v, sem):
        wid = lax.axis_index("s") * NC + lax.axis_index("c")
        base = wid * per_w

        def chunk(ci, carry):
            off = base + ci * nb
            pltpu.sync_copy(idx_hbm.at[pl.ds(off, nb)], idx_v)
            pltpu.async_copy(table_hbm.at[idx_v], rows_v, sem).wait()
            pltpu.sync_copy(rows_v, out_hbm.at[pl.ds(off, nb)])
            return carry

        lax.fori_loop(0, n_chunks, chunk, 0)

    return k(table, idx)


def _gather_mean2(table_a, table_b, idx_flat, n_chunks):
    NC, NS = _sc_geometry()
    NW = NC * NS
    F = 11
    D = table_a.shape[1]
    Nn = idx_flat.shape[0] // F
    per_w = Nn // NW
    nb = per_w // n_chunks
    mesh = plsc.VectorSubcoreMesh(core_axis_name="c", subcore_axis_name="s",
                                  num_cores=NC, num_subcores=NS)
    out_sd = jax.ShapeDtypeStruct((Nn, D), table_a.dtype)

    @functools.partial(
        pl.kernel, mesh=mesh,
        out_type=(out_sd, out_sd),
        scratch_types=[
            pltpu.VMEM((nb * F,), jnp.int32),
            pltpu.VMEM((nb * F, D), table_a.dtype),
            pltpu.VMEM((nb, D), table_a.dtype),
            pltpu.SemaphoreType.DMA,
        ],
    )
    def k(ta_hbm, tb_hbm, idx_hbm, oa_hbm, ob_hbm, idx_v, rows_v, out_v, sem):
        wid = lax.axis_index("s") * NC + lax.axis_index("c")
        base = wid * per_w

        def reduce_mean(_):
            def node(n, carry):
                r0 = n * F
                for c in range(D // 16):
                    sl = pl.ds(c * 16, 16)
                    acc = rows_v[r0, sl]
                    for j in range(1, F):
                        acc = acc + rows_v[r0 + j, sl]
                    out_v[n, sl] = acc * (1.0 / F)
                return carry
            lax.fori_loop(0, nb, node, 0)

        def chunk(ci, carry):
            off = base + ci * nb
            pltpu.sync_copy(idx_hbm.at[pl.ds(off * F, nb * F)], idx_v)
            pltpu.async_copy(ta_hbm.at[idx_v], rows_v, sem).wait()
            reduce_mean(None)
            pltpu.sync_copy(out_v, oa_hbm.at[pl.ds(off, nb)])
            pltpu.async_copy(tb_hbm.at[idx_v], rows_v, sem).wait()
            reduce_mean(None)
            pltpu.sync_copy(out_v, ob_hbm.at[pl.ds(off, nb)])
            return carry

        lax.fori_loop(0, n_chunks, chunk, 0)

    return k(table_a, table_b, idx_flat)


def _batch_gather_mean2(ta, tb, nodes_p, neigh_rows, S, n_chunks, nb):
    NC, NS = _sc_geometry()
    NW = NC * NS
    H = ta.shape[1]
    Bp = nodes_p.shape[0]
    per_w = Bp // NW
    assert per_w == n_chunks * nb and nb % 16 == 0
    mesh = plsc.VectorSubcoreMesh(core_axis_name="c", subcore_axis_name="s",
                                  num_cores=NC, num_subcores=NS)
    out_sd = jax.ShapeDtypeStruct((Bp, H), jnp.float32)
    F = S + 1

    @functools.partial(
        pl.kernel, mesh=mesh,
        out_type=(out_sd, out_sd),
        scratch_types=[
            pltpu.VMEM((nb,), jnp.int32),
            pltpu.VMEM((2 * nb,), jnp.int32),
            pltpu.VMEM((2 * nb, 128), jnp.int32),
            pltpu.VMEM((nb * S,), jnp.int32),
            pltpu.VMEM((nb * S, H), jnp.float32),
            pltpu.VMEM((nb, H), jnp.float32),
            pltpu.VMEM((nb, H), jnp.float32),
            pltpu.SemaphoreType.DMA,
        ],
        compiler_params=pltpu.CompilerParams(needs_layout_passes=False),
    )
    def k(ta_hbm, tb_hbm, nodes_hbm, nrows_hbm, oa_hbm, ob_hbm,
          nodes_v, ridx_v, irows_v, fidx_v, rows_v, self_v, out_v, sem):
        wid = lax.axis_index("s") * NC + lax.axis_index("c")
        base = wid * per_w
        lanes = lax.iota(jnp.int32, 16)

        def reduce_mean(_):
            def node(n, carry):
                r0 = n * S
                for c in range(H // 16):
                    sl = pl.ds(c * 16, 16)
                    acc = self_v[n, sl]
                    for j in range(S):
                        acc = acc + rows_v[r0 + j, sl]
                    out_v[n, sl] = acc * (1.0 / F)
                return carry
            lax.fori_loop(0, nb, node, 0)

        def chunk(ci, carry):
            off = base + ci * nb
            pltpu.sync_copy(nodes_hbm.at[pl.ds(off, nb)], nodes_v)
            for t in range(nb // 16):
                g = nodes_v[pl.ds(t * 16, 16)]
                p = lax.shift_right_logical(g * S, 7)
                pos = (lanes + t * 16) * 2
                plsc.store_scatter(ridx_v, [pos], p)
                plsc.store_scatter(ridx_v, [pos + 1], p + 1)
            pltpu.async_copy(nrows_hbm.at[ridx_v], irows_v, sem).wait()
            for t in range(nb // 16):
                g = nodes_v[pl.ds(t * 16, 16)]
                o = (g * S) & 127
                w = lanes + t * 16
                for j in range(S):
                    oj = o + j
                    row = w * 2 + lax.shift_right_logical(oj, 7)
                    col = oj & 127
                    vals = plsc.load_gather(irows_v, [row, col])
                    plsc.store_scatter(fidx_v, [w * S + j], vals)
            pltpu.async_copy(ta_hbm.at[fidx_v], rows_v, sem).wait()
            pltpu.async_copy(ta_hbm.at[nodes_v], self_v, sem).wait()
            reduce_mean(None)
            pltpu.sync_copy(out_v, oa_hbm.at[pl.ds(off, nb)])
            pltpu.async_copy(tb_hbm.at[fidx_v], rows_v, sem).wait()
            pltpu.async_copy(tb_hbm.at[nodes_v], self_v, sem).wait()
            reduce_mean(None)
            pltpu.sync_copy(out_v, ob_hbm.at[pl.ds(off, nb)])
            return carry

        lax.fori_loop(0, n_chunks, chunk, 0)

    return k(ta, tb, nodes_p, neigh_rows)


def _mm_relu2(xa, xb, w, bm):
    M, K = xa.shape
    H = w.shape[0]

    def body(xa_ref, xb_ref, w_ref, oa_ref, ob_ref):
        dn = (((1,), (1,)), ((), ()))
        oa_ref[...] = jnp.maximum(
            lax.dot_general(xa_ref[...], w_ref[...], dn,
                            preferred_element_type=jnp.float32), 0.0)
        ob_ref[...] = jnp.maximum(
            lax.dot_general(xb_ref[...], w_ref[...], dn,
                            preferred_element_type=jnp.float32), 0.0)

    out_sd = jax.ShapeDtypeStruct((M, H), jnp.float32)
    return pl.pallas_call(
        body,
        grid=(M // bm,),
        in_specs=[
            pl.BlockSpec((bm, K), lambda i: (i, 0)),
            pl.BlockSpec((bm, K), lambda i: (i, 0)),
            pl.BlockSpec((H, K), lambda i: (0, 0)),
        ],
        out_specs=[
            pl.BlockSpec((bm, H), lambda i: (i, 0)),
            pl.BlockSpec((bm, H), lambda i: (i, 0)),
        ],
        out_shape=(out_sd, out_sd),
    )(xa, xb, w)


def _mm_relu2_sum(xa, xb, w, mcol, bm):
    M, K = xa.shape
    H = w.shape[0]

    def body(xa_ref, xb_ref, w_ref, m_ref, oa_ref, ob_ref, s_ref):
        dn = (((1,), (1,)), ((), ()))
        ha = jnp.maximum(
            lax.dot_general(xa_ref[...], w_ref[...], dn,
                            preferred_element_type=jnp.float32), 0.0)
        hb = jnp.maximum(
            lax.dot_general(xb_ref[...], w_ref[...], dn,
                            preferred_element_type=jnp.float32), 0.0)
        oa_ref[...] = ha
        ob_ref[...] = hb

        @pl.when(pl.program_id(0) == 0)
        def _():
            s_ref[...] = jnp.zeros_like(s_ref)

        s_ref[...] += jnp.sum(ha * m_ref[...], axis=0, keepdims=True)

    out_sd = jax.ShapeDtypeStruct((M, H), jnp.float32)
    s_sd = jax.ShapeDtypeStruct((1, H), jnp.float32)
    return pl.pallas_call(
        body,
        grid=(M // bm,),
        in_specs=[
            pl.BlockSpec((bm, K), lambda i: (i, 0)),
            pl.BlockSpec((bm, K), lambda i: (i, 0)),
            pl.BlockSpec((H, K), lambda i: (0, 0)),
            pl.BlockSpec((bm, 1), lambda i: (i, 0)),
        ],
        out_specs=[
            pl.BlockSpec((bm, H), lambda i: (i, 0)),
            pl.BlockSpec((bm, H), lambda i: (i, 0)),
            pl.BlockSpec((1, H), lambda i: (0, 0)),
        ],
        out_shape=(out_sd, out_sd, s_sd),
    )(xa, xb, w, mcol)


def _readout_scores(s, msk, wd, bd11, b1, b2, h2, h2c):
    B, H = h2.shape

    def body(s_ref, m_ref, wd_ref, bd_ref, b1_ref, b2_ref, h_ref, hc_ref,
             o1_ref, o2_ref):
        summ = jnp.sum(m_ref[...])
        c = jax.nn.sigmoid(s_ref[...] / summ)
        dn = (((1,), (1,)), ((), ()))
        cw = lax.dot_general(c, wd_ref[...], dn,
                             preferred_element_type=jnp.float32)
        bd = bd_ref[0, 0]
        o1_ref[...] = lax.dot_general(cw, h_ref[...], dn,
                                      preferred_element_type=jnp.float32) \
            + bd + b1_ref[...]
        o2_ref[...] = lax.dot_general(cw, hc_ref[...], dn,
                                      preferred_element_type=jnp.float32) \
            + bd + b2_ref[...]

    o_sd = jax.ShapeDtypeStruct((1, B), jnp.float32)
    return pl.pallas_call(body, out_shape=(o_sd, o_sd))(
        s, msk, wd, bd11, b1, b2, h2, h2c)


def kernel(features, msk, samp_bias1, samp_bias2, W1, W2, Wd, bd, neigh,
           nodes, perm):
    N, D = features.shape
    S = neigh.shape[1]
    B = nodes.shape[0]
    H = W1.shape[0]
    F = S + 1

    NC, NS = _sc_geometry()
    NW = NC * NS
    ALIGN = 8 * NW

    def pad_to(x, m):
        return ((x + m - 1) // m) * m

    Np = pad_to(N, ALIGN)
    Bp = pad_to(B, ALIGN)

    self_idx = jnp.arange(N, dtype=neigh.dtype)[:, None]
    idx1 = jnp.concatenate([self_idx, neigh], axis=1).reshape(-1)
    idx1 = jnp.pad(idx1, (0, Np * F - N * F))
    perm_p = jnp.pad(perm, (0, Np - N))
    nodes_p = jnp.pad(nodes, (0, Bp - B))

    R128 = (N * S + 127) // 128 + 1
    neigh_rows = jnp.pad(neigh.reshape(-1), (0, R128 * 128 - N * S)) \
        .reshape(R128, 128)

    pf = _row_gather(features, perm_p, n_chunks=4)

    agg1, agg1c = _gather_mean2(features, pf, idx1, n_chunks=28)

    h1, h1c = _mm_relu2(agg1, agg1c, W1, bm=512)

    agg2, agg2c = _batch_gather_mean2(h1, h1c, nodes_p, neigh_rows, S,
                                      n_chunks=10, nb=32)

    mcol = jnp.pad(msk.reshape(B, 1), ((0, Bp - B), (0, 0)))
    h2, h2c, s = _mm_relu2_sum(agg2, agg2c, W2, mcol, bm=512)

    bd11 = jnp.reshape(bd, (1, 1)).astype(jnp.float32)
    sc_1, sc_2 = _readout_scores(s, msk, Wd, bd11, samp_bias1, samp_bias2,
                                 h2[:B], h2c[:B])
    return jnp.concatenate([sc_1, sc_2], axis=1)


# speedup vs baseline: 5.4619x; 1.5458x over previous
import functools

import jax
import jax.numpy as jnp
from jax import lax
from jax.experimental import pallas as pl
from jax.experimental.pallas import tpu as pltpu
from jax.experimental.pallas import tpu_sc as plsc


def _sc_geometry():
    try:
        info = plsc.get_sparse_core_info()
        return info.num_cores, info.num_subcores
    except Exception:
        return 2, 16


_SC_PARAMS = pltpu.CompilerParams(needs_layout_passes=False)
_HI_MASK = -65536


def _unpack_pair(w):
    va = plsc.bitcast(lax.shift_left(w, 16), jnp.float32)
    vb = plsc.bitcast(lax.bitwise_and(w, _HI_MASK), jnp.float32)
    return va, vb


def _build_pair_table(fpad, perm_p, n_chunks):
    NC, NS = _sc_geometry()
    NW = NC * NS
    Np, D = fpad.shape
    per_w = Np // NW
    nb = per_w // n_chunks
    mesh = plsc.VectorSubcoreMesh(core_axis_name="c", subcore_axis_name="s",
                                  num_cores=NC, num_subcores=NS)

    @functools.partial(
        pl.kernel, mesh=mesh,
        out_type=jax.ShapeDtypeStruct((Np, D), jnp.int32),
        scratch_types=[
            pltpu.VMEM((nb,), jnp.int32),
            pltpu.VMEM((nb, D), jnp.float32),
            pltpu.VMEM((nb, D), jnp.float32),
            pltpu.VMEM((nb, D), jnp.int32),
            pltpu.SemaphoreType.DMA,
        ],
        compiler_params=_SC_PARAMS,
    )
    def k(f_hbm, perm_hbm, out_hbm, perm_v, selfb_v, gath_v, out_v, sem):
        wid = lax.axis_index("s") * NC + lax.axis_index("c")
        base = wid * per_w

        def chunk(ci, carry):
            off = base + ci * nb
            pltpu.sync_copy(f_hbm.at[pl.ds(off, nb)], selfb_v)
            pltpu.sync_copy(perm_hbm.at[pl.ds(off, nb)], perm_v)
            pltpu.async_copy(f_hbm.at[perm_v], gath_v, sem).wait()

            def row(n, carry2):
                for c in range(D // 16):
                    sl = pl.ds(c * 16, 16)
                    packed = plsc.pack(selfb_v[n, sl], gath_v[n, sl],
                                       format=plsc.PackFormat.INTERLEAVED)
                    out_v[n, sl] = plsc.bitcast(packed, jnp.int32)
                return carry2

            lax.fori_loop(0, nb, row, 0)
            pltpu.sync_copy(out_v, out_hbm.at[pl.ds(off, nb)])
            return carry

        lax.fori_loop(0, n_chunks, chunk, 0)

    return k(fpad, perm_p)


def _gather_mean_pair(tpair, idx_flat, n_chunks):
    NC, NS = _sc_geometry()
    NW = NC * NS
    F = 11
    D = tpair.shape[1]
    Nn = idx_flat.shape[0] // F
    per_w = Nn // NW
    nb = per_w // n_chunks
    mesh = plsc.VectorSubcoreMesh(core_axis_name="c", subcore_axis_name="s",
                                  num_cores=NC, num_subcores=NS)
    out_sd = jax.ShapeDtypeStruct((Nn, D), jnp.float32)

    @functools.partial(
        pl.kernel, mesh=mesh,
        out_type=(out_sd, out_sd),
        scratch_types=[
            pltpu.VMEM((nb * F,), jnp.int32),
            pltpu.VMEM((nb * F, D), jnp.int32),
            pltpu.VMEM((nb, D), jnp.float32),
            pltpu.VMEM((nb, D), jnp.float32),
            pltpu.SemaphoreType.DMA,
        ],
        compiler_params=_SC_PARAMS,
    )
    def k(t_hbm, idx_hbm, oa_hbm, ob_hbm, idx_v, rows_v, oa_v, ob_v, sem):
        wid = lax.axis_index("s") * NC + lax.axis_index("c")
        base = wid * per_w

        def reduce_mean(_):
            def node(n, carry):
                r0 = n * F
                for c in range(D // 16):
                    sl = pl.ds(c * 16, 16)
                    ea, eb = _unpack_pair(rows_v[r0, sl])
                    for j in range(1, F):
                        ja, jb = _unpack_pair(rows_v[r0 + j, sl])
                        ea = ea + ja
                        eb = eb + jb
                    oa_v[n, sl] = ea * (1.0 / F)
                    ob_v[n, sl] = eb * (1.0 / F)
                return carry
            lax.fori_loop(0, nb, node, 0)

        def chunk(ci, carry):
            off = base + ci * nb
            pltpu.sync_copy(idx_hbm.at[pl.ds(off * F, nb * F)], idx_v)
            pltpu.async_copy(t_hbm.at[idx_v], rows_v, sem).wait()
            reduce_mean(None)
            pltpu.sync_copy(oa_v, oa_hbm.at[pl.ds(off, nb)])
            pltpu.sync_copy(ob_v, ob_hbm.at[pl.ds(off, nb)])
            return carry

        lax.fori_loop(0, n_chunks, chunk, 0)

    return k(tpair, idx_flat)


def _batch_gather_mean_pair(tpair, nodes_p, neigh_rows, S, n_chunks, nb):
    NC, NS = _sc_geometry()
    NW = NC * NS
    H = tpair.shape[1]
    Bp = nodes_p.shape[0]
    per_w = Bp // NW
    assert per_w == n_chunks * nb and nb % 16 == 0
    mesh = plsc.VectorSubcoreMesh(core_axis_name="c", subcore_axis_name="s",
                                  num_cores=NC, num_subcores=NS)
    out_sd = jax.ShapeDtypeStruct((Bp, H), jnp.float32)
    F = S + 1

    @functools.partial(
        pl.kernel, mesh=mesh,
        out_type=(out_sd, out_sd),
        scratch_types=[
            pltpu.VMEM((nb,), jnp.int32),
            pltpu.VMEM((2 * nb,), jnp.int32),
            pltpu.VMEM((2 * nb, 128), jnp.int32),
            pltpu.VMEM((nb * S,), jnp.int32),
            pltpu.VMEM((nb * S, H), jnp.int32),
            pltpu.VMEM((nb, H), jnp.int32),
            pltpu.VMEM((nb, H), jnp.float32),
            pltpu.VMEM((nb, H), jnp.float32),
            pltpu.SemaphoreType.DMA,
        ],
        compiler_params=_SC_PARAMS,
    )
    def k(t_hbm, nodes_hbm, nrows_hbm, oa_hbm, ob_hbm,
          nodes_v, ridx_v, irows_v, fidx_v, rows_v, self_v, oa_v, ob_v, sem):
        wid = lax.axis_index("s") * NC + lax.axis_index("c")
        base = wid * per_w
        lanes = lax.iota(jnp.int32, 16)

        def reduce_mean(_):
            def node(n, carry):
                r0 = n * S
                for c in range(H // 16):
                    sl = pl.ds(c * 16, 16)
                    ea, eb = _unpack_pair(self_v[n, sl])
                    for j in range(S):
                        ja, jb = _unpack_pair(rows_v[r0 + j, sl])
                        ea = ea + ja
                        eb = eb + jb
                    oa_v[n, sl] = ea * (1.0 / F)
                    ob_v[n, sl] = eb * (1.0 / F)
                return carry
            lax.fori_loop(0, nb, node, 0)

        def chunk(ci, carry):
            off = base + ci * nb
            pltpu.sync_copy(nodes_hbm.at[pl.ds(off, nb)], nodes_v)
            for t in range(nb // 16):
                g = nodes_v[pl.ds(t * 16, 16)]
                p = lax.shift_right_logical(g * S, 7)
                pos = (lanes + t * 16) * 2
                plsc.store_scatter(ridx_v, [pos], p)
                plsc.store_scatter(ridx_v, [pos + 1], p + 1)
            pltpu.async_copy(nrows_hbm.at[ridx_v], irows_v, sem).wait()
            for t in range(nb // 16):
                g = nodes_v[pl.ds(t * 16, 16)]
                o = (g * S) & 127
                w = lanes + t * 16
                for j in range(S):
                    oj = o + j
                    row = w * 2 + lax.shift_right_logical(oj, 7)
                    col = oj & 127
                    vals = plsc.load_gather(irows_v, [row, col])
                    plsc.store_scatter(fidx_v, [w * S + j], vals)
            pltpu.async_copy(t_hbm.at[fidx_v], rows_v, sem).wait()
            pltpu.async_copy(t_hbm.at[nodes_v], self_v, sem).wait()
            reduce_mean(None)
            pltpu.sync_copy(oa_v, oa_hbm.at[pl.ds(off, nb)])
            pltpu.sync_copy(ob_v, ob_hbm.at[pl.ds(off, nb)])
            return carry

        lax.fori_loop(0, n_chunks, chunk, 0)

    return k(tpair, nodes_p, neigh_rows)


def _mm_relu2_pack(xa, xb, w, bm):
    M, K = xa.shape
    H = w.shape[0]

    def body(xa_ref, xb_ref, w_ref, o_ref):
        dn = (((1,), (1,)), ((), ()))
        ha = jnp.maximum(
            lax.dot_general(xa_ref[...], w_ref[...], dn,
                            preferred_element_type=jnp.float32), 0.0)
        hb = jnp.maximum(
            lax.dot_general(xb_ref[...], w_ref[...], dn,
                            preferred_element_type=jnp.float32), 0.0)
        a16 = lax.bitcast_convert_type(
            ha.astype(jnp.bfloat16), jnp.uint16).astype(jnp.uint32)
        b16 = lax.bitcast_convert_type(
            hb.astype(jnp.bfloat16), jnp.uint16).astype(jnp.uint32)
        packed = lax.bitwise_or(a16, lax.shift_left(b16, jnp.uint32(16)))
        o_ref[...] = lax.bitcast_convert_type(packed, jnp.int32)

    out_sd = jax.ShapeDtypeStruct((M, H), jnp.int32)
    return pl.pallas_call(
        body,
        grid=(M // bm,),
        in_specs=[
            pl.BlockSpec((bm, K), lambda i: (i, 0)),
            pl.BlockSpec((bm, K), lambda i: (i, 0)),
            pl.BlockSpec((H, K), lambda i: (0, 0)),
        ],
        out_specs=pl.BlockSpec((bm, H), lambda i: (i, 0)),
        out_shape=out_sd,
    )(xa, xb, w)


def _mm_relu2_sum(xa, xb, w, mcol, bm):
    M, K = xa.shape
    H = w.shape[0]

    def body(xa_ref, xb_ref, w_ref, m_ref, oa_ref, ob_ref, s_ref):
        dn = (((1,), (1,)), ((), ()))
        ha = jnp.maximum(
            lax.dot_general(xa_ref[...], w_ref[...], dn,
                            preferred_element_type=jnp.float32), 0.0)
        hb = jnp.maximum(
            lax.dot_general(xb_ref[...], w_ref[...], dn,
                            preferred_element_type=jnp.float32), 0.0)
        oa_ref[...] = ha
        ob_ref[...] = hb

        @pl.when(pl.program_id(0) == 0)
        def _():
            s_ref[...] = jnp.zeros_like(s_ref)

        s_ref[...] += jnp.sum(ha * m_ref[...], axis=0, keepdims=True)

    out_sd = jax.ShapeDtypeStruct((M, H), jnp.float32)
    s_sd = jax.ShapeDtypeStruct((1, H), jnp.float32)
    return pl.pallas_call(
        body,
        grid=(M // bm,),
        in_specs=[
            pl.BlockSpec((bm, K), lambda i: (i, 0)),
            pl.BlockSpec((bm, K), lambda i: (i, 0)),
            pl.BlockSpec((H, K), lambda i: (0, 0)),
            pl.BlockSpec((bm, 1), lambda i: (i, 0)),
        ],
        out_specs=[
            pl.BlockSpec((bm, H), lambda i: (i, 0)),
            pl.BlockSpec((bm, H), lambda i: (i, 0)),
            pl.BlockSpec((1, H), lambda i: (0, 0)),
        ],
        out_shape=(out_sd, out_sd, s_sd),
    )(xa, xb, w, mcol)


def _readout_scores(s, msk, wd, bd11, b1, b2, h2, h2c):
    B, H = h2.shape

    def body(s_ref, m_ref, wd_ref, bd_ref, b1_ref, b2_ref, h_ref, hc_ref,
             o1_ref, o2_ref):
        summ = jnp.sum(m_ref[...])
        c = jax.nn.sigmoid(s_ref[...] / summ)
        dn = (((1,), (1,)), ((), ()))
        cw = lax.dot_general(c, wd_ref[...], dn,
                             preferred_element_type=jnp.float32)
        bd = bd_ref[0, 0]
        o1_ref[...] = lax.dot_general(cw, h_ref[...], dn,
                                      preferred_element_type=jnp.float32) \
            + bd + b1_ref[...]
        o2_ref[...] = lax.dot_general(cw, hc_ref[...], dn,
                                      preferred_element_type=jnp.float32) \
            + bd + b2_ref[...]

    o_sd = jax.ShapeDtypeStruct((1, B), jnp.float32)
    return pl.pallas_call(body, out_shape=(o_sd, o_sd))(
        s, msk, wd, bd11, b1, b2, h2, h2c)


def kernel(features, msk, samp_bias1, samp_bias2, W1, W2, Wd, bd, neigh,
           nodes, perm):
    N, D = features.shape
    S = neigh.shape[1]
    B = nodes.shape[0]
    H = W1.shape[0]
    F = S + 1

    NC, NS = _sc_geometry()
    NW = NC * NS
    ALIGN = 8 * NW

    def pad_to(x, m):
        return ((x + m - 1) // m) * m

    Np = pad_to(N, ALIGN)
    Bp = pad_to(B, ALIGN)

    self_idx = jnp.arange(N, dtype=neigh.dtype)[:, None]
    idx1 = jnp.concatenate([self_idx, neigh], axis=1).reshape(-1)
    idx1 = jnp.pad(idx1, (0, Np * F - N * F))
    perm_p = jnp.pad(perm, (0, Np - N))
    nodes_p = jnp.pad(nodes, (0, Bp - B))
    fpad = jnp.pad(features, ((0, Np - N), (0, 0)))

    R128 = (N * S + 127) // 128 + 1
    neigh_rows = jnp.pad(neigh.reshape(-1), (0, R128 * 128 - N * S)) \
        .reshape(R128, 128)

    t1 = _build_pair_table(fpad, perm_p, n_chunks=14)

    agg1, agg1c = _gather_mean_pair(t1, idx1, n_chunks=28)

    t2 = _mm_relu2_pack(agg1, agg1c, W1, bm=512)

    agg2, agg2c = _batch_gather_mean_pair(t2, nodes_p, neigh_rows, S,
                                          n_chunks=10, nb=32)

    mcol = jnp.pad(msk.reshape(B, 1), ((0, Bp - B), (0, 0)))
    h2, h2c, s = _mm_relu2_sum(agg2, agg2c, W2, mcol, bm=512)

    bd11 = jnp.reshape(bd, (1, 1)).astype(jnp.float32)
    sc_1, sc_2 = _readout_scores(s, msk, Wd, bd11, samp_bias1, samp_bias2,
                                 h2[:B], h2c[:B])
    return jnp.concatenate([sc_1, sc_2], axis=1)
